# Initial kernel scaffold; baseline (speedup 1.0000x reference)
import jax
import jax.numpy as jnp
from jax.experimental import pallas as pl


def kernel(word, word_table, pos_table):
    raise NotImplementedError("write your pallas kernel here")


# speedup vs baseline: 9.7268x; 9.7268x over previous
import functools

import jax
import jax.numpy as jnp
from jax import lax
from jax.experimental import pallas as pl
from jax.experimental.pallas import tpu as pltpu
from jax.experimental.pallas import tpu_sc as plsc

INSN = 12
NINSN = 10
SEQ = 120
D = 64
OUT_PER_ROW = 3 * NINSN

NC, NS = 2, 16
NW = NC * NS

B = 4096
ROWS_PER_W = B // NW
CHUNK = 8
STEPS = ROWS_PER_W // CHUNK
CIDX = CHUNK * SEQ
COUT = CHUNK * OUT_PER_ROW


def _body(idx_hbm, table_hbm, pos_hbm, out_hbm, idx_v, rows_v, out_v, pos_v, sem):
    wid = lax.axis_index("s") * NC + lax.axis_index("c")
    pltpu.sync_copy(pos_hbm, pos_v)
    base0 = wid * ROWS_PER_W

    @pl.loop(0, STEPS)
    def step(s):
        base = base0 + s * CHUNK
        pltpu.sync_copy(idx_hbm.at[pl.ds(base * SEQ, CIDX)], idx_v)
        pltpu.async_copy(table_hbm.at[idx_v], rows_v, sem).wait()

        @pl.loop(0, CHUNK)
        def row(r):
            for j in range(NINSN):
                ro = r * SEQ + j * INSN
                oo = r * OUT_PER_ROW + j * 3
                for q in range(D // 16):
                    sl = pl.ds(q * 16, 16)
                    p = pos_v[j, sl]
                    a1 = rows_v[ro + 0, sl] + rows_v[ro + 1, sl]
                    a2 = ((rows_v[ro + 2, sl] + rows_v[ro + 3, sl])
                          + (rows_v[ro + 4, sl] + rows_v[ro + 5, sl])
                          + rows_v[ro + 6, sl])
                    a3 = ((rows_v[ro + 7, sl] + rows_v[ro + 8, sl])
                          + (rows_v[ro + 9, sl] + rows_v[ro + 10, sl])
                          + rows_v[ro + 11, sl])
                    out_v[oo + 0, sl] = a1 + p
                    out_v[oo + 1, sl] = a2 + p
                    out_v[oo + 2, sl] = a3 + p

        pltpu.sync_copy(out_v, out_hbm.at[pl.ds(base * OUT_PER_ROW, COUT)])


@jax.jit
def _run(idx_flat, word_table, pos_table):
    mesh = plsc.VectorSubcoreMesh(
        core_axis_name="c", subcore_axis_name="s", num_cores=NC, num_subcores=NS)
    k = pl.kernel(
        _body,
        out_type=jax.ShapeDtypeStruct((B * OUT_PER_ROW, D), jnp.float32),
        mesh=mesh,
        scratch_types=[
            pltpu.VMEM((CIDX,), jnp.int32),
            pltpu.VMEM((CIDX, D), jnp.float32),
            pltpu.VMEM((COUT, D), jnp.float32),
            pltpu.VMEM((NINSN, D), jnp.float32),
            pltpu.SemaphoreType.DMA,
        ],
        compiler_params=pltpu.CompilerParams(use_tc_tiling_on_sc=False),
    )
    return k(idx_flat, word_table, pos_table)


def kernel(word, word_table, pos_table):
    idx_flat = word.reshape(-1).astype(jnp.int32)
    pos10 = lax.slice_in_dim(pos_table, 1, 1 + NINSN, axis=0)
    out = _run(idx_flat, word_table, pos10)
    return out.reshape(-1, OUT_PER_ROW, D)


# speedup vs baseline: 10.6712x; 1.0971x over previous
import jax
import jax.numpy as jnp
from jax import lax
from jax.experimental import pallas as pl
from jax.experimental.pallas import tpu as pltpu
from jax.experimental.pallas import tpu_sc as plsc

INSN = 12
NINSN = 10
SEQ = 120
D = 64
OUT_PER_ROW = 3 * NINSN

NC, NS = 2, 16
NW = NC * NS

B = 4096
ROWS_PER_W = B // NW
CHUNK = 4
STEPS = ROWS_PER_W // CHUNK
CIDX = CHUNK * SEQ
COUT = CHUNK * OUT_PER_ROW


def _compute(rows_v, pos_v, out_v):
    @pl.loop(0, CHUNK)
    def row(r):
        for j in range(NINSN):
            ro = r * SEQ + j * INSN
            oo = r * OUT_PER_ROW + j * 3
            for q in range(D // 16):
                sl = pl.ds(q * 16, 16)
                p = pos_v[j, sl]
                a1 = rows_v[ro + 0, sl] + rows_v[ro + 1, sl]
                a2 = ((rows_v[ro + 2, sl] + rows_v[ro + 3, sl])
                      + (rows_v[ro + 4, sl] + rows_v[ro + 5, sl])
                      + rows_v[ro + 6, sl])
                a3 = ((rows_v[ro + 7, sl] + rows_v[ro + 8, sl])
                      + (rows_v[ro + 9, sl] + rows_v[ro + 10, sl])
                      + rows_v[ro + 11, sl])
                out_v[oo + 0, sl] = a1 + p
                out_v[oo + 1, sl] = a2 + p
                out_v[oo + 2, sl] = a3 + p


def _body(idx_hbm, table_hbm, pos_hbm, out_hbm,
          idx_v0, idx_v1, rows_v0, rows_v1, out_v0, out_v1, pos_v,
          gsem0, gsem1, osem0, osem1):
    wid = lax.axis_index("s") * NC + lax.axis_index("c")
    pltpu.sync_copy(pos_hbm, pos_v)
    base0 = wid * ROWS_PER_W

    idx_v = (idx_v0, idx_v1)
    rows_v = (rows_v0, rows_v1)
    out_v = (out_v0, out_v1)
    gsem = (gsem0, gsem1)
    osem = (osem0, osem1)

    pltpu.sync_copy(idx_hbm.at[pl.ds(base0 * SEQ, CIDX)], idx_v[0])
    pltpu.async_copy(table_hbm.at[idx_v[0]], rows_v[0], gsem[0])

    @pl.loop(0, STEPS // 2)
    def pair(ps):
        for b in range(2):
            s = ps * 2 + b
            nb = 1 - b
            base = base0 + s * CHUNK

            @pl.when(s + 1 < STEPS)
            def _():
                nbase = base + CHUNK
                pltpu.sync_copy(idx_hbm.at[pl.ds(nbase * SEQ, CIDX)], idx_v[nb])
                pltpu.async_copy(table_hbm.at[idx_v[nb]], rows_v[nb], gsem[nb])

            pltpu.make_async_copy(
                table_hbm.at[idx_v[b]], rows_v[b], gsem[b]).wait()

            @pl.when(s >= 2)
            def _():
                pltpu.make_async_copy(
                    out_v[b], out_hbm.at[pl.ds(base * OUT_PER_ROW, COUT)],
                    osem[b]).wait()

            _compute(rows_v[b], pos_v, out_v[b])
            pltpu.async_copy(
                out_v[b], out_hbm.at[pl.ds(base * OUT_PER_ROW, COUT)], osem[b])

    for b in range(2):
        pltpu.make_async_copy(
            out_v[b], out_hbm.at[pl.ds(base0 * OUT_PER_ROW, COUT)],
            osem[b]).wait()


@jax.jit
def _run(idx_flat, word_table, pos_table):
    mesh = plsc.VectorSubcoreMesh(
        core_axis_name="c", subcore_axis_name="s", num_cores=NC, num_subcores=NS)
    k = pl.kernel(
        _body,
        out_type=jax.ShapeDtypeStruct((B * OUT_PER_ROW, D), jnp.float32),
        mesh=mesh,
        scratch_types=[
            pltpu.VMEM((CIDX,), jnp.int32),
            pltpu.VMEM((CIDX,), jnp.int32),
            pltpu.VMEM((CIDX, D), jnp.float32),
            pltpu.VMEM((CIDX, D), jnp.float32),
            pltpu.VMEM((COUT, D), jnp.float32),
            pltpu.VMEM((COUT, D), jnp.float32),
            pltpu.VMEM((NINSN, D), jnp.float32),
            pltpu.SemaphoreType.DMA,
            pltpu.SemaphoreType.DMA,
            pltpu.SemaphoreType.DMA,
            pltpu.SemaphoreType.DMA,
        ],
        compiler_params=pltpu.CompilerParams(use_tc_tiling_on_sc=False),
    )
    return k(idx_flat, word_table, pos_table)


def kernel(word, word_table, pos_table):
    idx_flat = word.reshape(-1).astype(jnp.int32)
    pos10 = lax.slice_in_dim(pos_table, 1, 1 + NINSN, axis=0)
    out = _run(idx_flat, word_table, pos10)
    return out.reshape(-1, OUT_PER_ROW, D)
